# baseline (speedup 1.0000x reference)
import jax
import jax.numpy as jnp
from jax.experimental import pallas as pl

N_CORES = 10000
N_QUBITS = 100000
CORE_CAP = 32
N_EDGES = 160000
D = 256
B = 4


def _mm_body(x_ref, w_ref, o_ref):
    o_ref[...] = jnp.dot(x_ref[...], w_ref[...], preferred_element_type=jnp.float32)


def _matmul(x, w):
    rows = x.shape[0]
    blk = 400
    grid = rows // blk
    return pl.pallas_call(
        _mm_body,
        grid=(grid,),
        in_specs=[
            pl.BlockSpec((blk, D), lambda i: (i, 0)),
            pl.BlockSpec((D, D), lambda i: (0, 0)),
        ],
        out_specs=pl.BlockSpec((blk, D), lambda i: (i, 0)),
        out_shape=jax.ShapeDtypeStruct((rows, D), jnp.float32),
    )(x, w)


def kernel(core_allocs, qubit_embs, dummy_qubit_emb, edge_index, edge_weight, W1, b1, W2, b2):
    n = N_CORES

    def unmixed(alloc):
        counts = jnp.zeros((n,), jnp.int32).at[alloc].add(1)
        seg = jnp.full((n, D), -jnp.inf, qubit_embs.dtype).at[alloc].max(qubit_embs)
        has_pad = counts < CORE_CAP
        return jnp.where(has_pad[:, None], jnp.maximum(seg, dummy_qubit_emb[None, :]), seg)

    pre_embs = jax.vmap(unmixed)(core_allocs)

    src, dst = edge_index[0], edge_index[1]
    loop = jnp.arange(n)
    s = jnp.concatenate([src, loop])
    d = jnp.concatenate([dst, loop])
    w = jnp.concatenate([edge_weight, jnp.ones((n,), edge_weight.dtype)])
    deg = jnp.zeros((n,), jnp.float32).at[d].add(w)
    dinv = jnp.where(deg > 0, jax.lax.rsqrt(deg), 0.0)
    norm = dinv[s] * w * dinv[d]

    def gcn(x_flat, W, b):
        h = _matmul(x_flat.reshape(B * n, D), W).reshape(B, n, D)

        def agg(hb):
            return jnp.zeros_like(hb).at[d].add(hb[s] * norm[:, None])

        return jax.nn.relu(jax.vmap(agg)(h) + b)

    x1 = gcn(pre_embs, W1, b1)
    x2 = gcn(x1, W2, b2)
    return x2
